# baseline (speedup 1.0000x reference)
import functools

import jax
import jax.numpy as jnp
from jax import lax
from jax.experimental import pallas as pl
from jax.experimental.pallas import tpu as pltpu
from jax.experimental.pallas import tpu_sc as plsc

N = 100000
D = 128
S = 256
EPS = 1e-5

NW = 32
ROWS_W = 3136
N_PAD = NW * ROWS_W
BLOCKR = 224
NBLK = ROWS_W // BLOCKR
CH = BLOCKR // 16
ST = S + 1
SUMW = ST * D
CNTW = ST * 16

_NEG_INF = float("-inf")


def _sc_pool(x_hbm, ids_hbm, psum_hbm, pmax_hbm, pcnt_hbm,
             xbuf, idbuf, sumtab, maxtab, cnttab):
    wid = lax.axis_index("s") * 2 + lax.axis_index("c")
    zero16 = jnp.zeros((16,), jnp.float32)
    ninf16 = jnp.full((16,), _NEG_INF, jnp.float32)
    one16 = jnp.ones((16,), jnp.float32)
    col = lax.iota(jnp.int32, 16)

    def initf(i, c):
        sumtab[pl.ds(i * 16, 16)] = zero16
        maxtab[pl.ds(i * 16, 16)] = ninf16
        return c

    lax.fori_loop(0, SUMW // 16, initf, 0)

    def initc(i, c):
        cnttab[pl.ds(i * 16, 16)] = zero16
        return c

    lax.fori_loop(0, CNTW // 16, initc, 0)

    row0 = wid * ROWS_W

    def block_body(b, carry):
        base = row0 + b * BLOCKR
        pltpu.sync_copy(x_hbm.at[pl.ds(base, BLOCKR), :], xbuf)
        pltpu.sync_copy(ids_hbm.at[pl.ds(base, BLOCKR)], idbuf)

        def chunk_body(c, cc):
            cb = c * 16
            ids_vec = idbuf[pl.ds(cb, 16)]
            plsc.addupdate_scatter(cnttab, [ids_vec * 16 + col], one16)
            for r in range(16):
                seg_b = plsc.load_gather(
                    idbuf, [jnp.full((16,), cb + r, jnp.int32)])
                abase = seg_b * D + col
                for j in range(8):
                    addr = abase + (j * 16)
                    v = xbuf[cb + r, pl.ds(j * 16, 16)]
                    plsc.addupdate_scatter(sumtab, [addr], v)
                    cur = plsc.load_gather(maxtab, [addr])
                    plsc.store_scatter(maxtab, [addr], jnp.maximum(cur, v))
            return cc

        lax.fori_loop(0, CH, chunk_body, 0)
        return carry

    lax.fori_loop(0, NBLK, block_body, 0)

    pltpu.sync_copy(sumtab, psum_hbm.at[wid])
    pltpu.sync_copy(maxtab, pmax_hbm.at[wid])
    pltpu.sync_copy(cnttab, pcnt_hbm.at[wid])


def _dot_t(a, b):
    return jax.lax.dot_general(a, b, (((1,), (1,)), ((), ())),
                               preferred_element_type=jnp.float32)


def _tc_tail(psum, pmax, pcnt, Wm, bm, Wx, bx, Ws, bs, Wgm, bgm, Wgx, bgx,
             Wgs, bgs, Wo, bo, gamma, beta, out_ref):
    sum_pool = jnp.zeros((S, D), jnp.float32)
    max_pool = jnp.full((S, D), _NEG_INF, jnp.float32)
    cnt2 = jnp.zeros((S, 16), jnp.float32)
    for i in range(NW):
        sum_pool = sum_pool + psum[i]
        max_pool = jnp.maximum(max_pool, pmax[i])
        cnt2 = cnt2 + pcnt[i]

    ones16 = jnp.ones((D, 16), dtype=jnp.float32)
    counts = lax.dot_general(cnt2, ones16, (((1,), (1,)), ((), ())),
                             preferred_element_type=jnp.float32)
    counts = jnp.maximum(counts, 1.0)
    mean_pool = sum_pool / counts

    mean_repr = _dot_t(mean_pool, Wm[...]) + bm[...]
    max_repr = _dot_t(max_pool, Wx[...]) + bx[...]
    sum_repr = _dot_t(sum_pool, Ws[...]) + bs[...]

    gm = _dot_t(mean_repr, Wgm[...]) + bgm[...]
    gx = _dot_t(max_repr, Wgx[...]) + bgx[...]
    gs = _dot_t(sum_repr, Wgs[...]) + bgs[...]
    gm = 1.0 / (1.0 + jnp.exp(-gm))
    gx = 1.0 / (1.0 + jnp.exp(-gx))
    gs = 1.0 / (1.0 + jnp.exp(-gs))

    mx = jnp.maximum(jnp.maximum(gm, gx), gs)
    em = jnp.exp(gm - mx)
    ex = jnp.exp(gx - mx)
    es = jnp.exp(gs - mx)
    denom = em + ex + es
    pooled = (em * mean_repr + ex * max_repr + es * sum_repr) / denom

    ge = _dot_t(pooled, Wo[...]) + bo[...]
    ones = jnp.ones((D, D), dtype=jnp.float32)
    mu = _dot_t(ge, ones) * (1.0 / D)
    dev = ge - mu
    var = _dot_t(dev * dev, ones) * (1.0 / D)
    out_ref[...] = dev / jnp.sqrt(var + EPS) * gamma[...] + beta[...]


def kernel(x, batch, Wm, bm, Wx, bx, Ws, bs, Wgm, bgm, Wgx, bgx,
           Wgs, bgs, Wo, bo, gamma, beta):
    ids = batch.astype(jnp.int32)
    x_pad = jnp.concatenate(
        [x, jnp.zeros((N_PAD - N, D), x.dtype)], axis=0)
    ids_pad = jnp.concatenate(
        [ids, jnp.full((N_PAD - N,), S, jnp.int32)])

    mesh = plsc.VectorSubcoreMesh(core_axis_name="c", subcore_axis_name="s")
    pool = functools.partial(
        pl.kernel,
        mesh=mesh,
        out_type=[
            jax.ShapeDtypeStruct((NW, SUMW), jnp.float32),
            jax.ShapeDtypeStruct((NW, SUMW), jnp.float32),
            jax.ShapeDtypeStruct((NW, CNTW), jnp.float32),
        ],
        scratch_types=[
            pltpu.VMEM((BLOCKR, D), jnp.float32),
            pltpu.VMEM((BLOCKR,), jnp.int32),
            pltpu.VMEM((SUMW,), jnp.float32),
            pltpu.VMEM((SUMW,), jnp.float32),
            pltpu.VMEM((CNTW,), jnp.float32),
        ],
        compiler_params=pltpu.CompilerParams(needs_layout_passes=False),
    )(_sc_pool)
    psum, pmax, pcnt = pool(x_pad, ids_pad)

    psum3 = psum.reshape(NW, ST, D)[:, :S, :]
    pmax3 = pmax.reshape(NW, ST, D)[:, :S, :]
    pcnt3 = pcnt.reshape(NW, ST, 16)[:, :S, :]

    b2 = lambda b: b.reshape(1, -1)
    Wgm_r = jnp.broadcast_to(Wgm, (D, D))
    Wgx_r = jnp.broadcast_to(Wgx, (D, D))
    Wgs_r = jnp.broadcast_to(Wgs, (D, D))
    bgm_r = jnp.broadcast_to(bgm.reshape(1, 1), (1, D))
    bgx_r = jnp.broadcast_to(bgx.reshape(1, 1), (1, D))
    bgs_r = jnp.broadcast_to(bgs.reshape(1, 1), (1, D))

    args = (psum3, pmax3, pcnt3, Wm, b2(bm), Wx, b2(bx), Ws, b2(bs),
            Wgm_r, bgm_r, Wgx_r, bgx_r, Wgs_r, bgs_r, Wo, b2(bo),
            b2(gamma), b2(beta))

    return pl.pallas_call(
        _tc_tail,
        out_shape=jax.ShapeDtypeStruct((S, D), jnp.float32),
    )(*args)


# speedup vs baseline: 3.7385x; 3.7385x over previous
import functools

import jax
import jax.numpy as jnp
from jax import lax
from jax.experimental import pallas as pl
from jax.experimental.pallas import tpu as pltpu
from jax.experimental.pallas import tpu_sc as plsc

N = 100000
D = 128
S = 256
EPS = 1e-5

NW = 32
ROWS_W = 3136
BLOCKR = 224
CH = BLOCKR // 16

_NEG_INF = float("-inf")


def _sc_pool(x_hbm, ids_hbm, psum_hbm, pmax_hbm, pcnt_hbm,
             xbuf, idbuf, sumtab, maxtab, cnttab):
    wid = lax.axis_index("s") * 2 + lax.axis_index("c")
    col = lax.iota(jnp.int32, 16)
    zero16 = jnp.zeros((16,), jnp.float32)

    def initc(i, c):
        cnttab[i, :] = zero16
        return c

    lax.fori_loop(0, S, initc, 0)

    r0 = wid * ROWS_W
    r1 = jnp.minimum(r0 + ROWS_W, N)
    nch = (r1 - r0) // 16
    nbt = (nch + CH - 1) // CH

    idbase = jnp.minimum(r0, N - ROWS_W)
    idoff = r0 - idbase
    pltpu.sync_copy(ids_hbm.at[pl.ds(idbase, ROWS_W)], idbuf)

    def row_fast(vrow, acc):
        sums, maxs, runseg, cnt = acc
        sums = tuple(sums[j] + vrow[j] for j in range(8))
        maxs = tuple(jnp.maximum(maxs[j], vrow[j]) for j in range(8))
        return sums, maxs, runseg, cnt + 1.0

    def row_slow(segr, vrow, acc):
        sums, maxs, runseg, cnt = acc
        eqv = segr == runseg
        flush = jnp.logical_and(jnp.logical_not(eqv), runseg >= 0)
        for j in range(8):
            cidx = col + (j * 16)
            plsc.store_scatter(sumtab, [runseg, cidx], sums[j], mask=flush)
            plsc.store_scatter(maxtab, [runseg, cidx], maxs[j], mask=flush)
        plsc.store_scatter(cnttab, [runseg, col], jnp.full((16,), cnt),
                           mask=flush)
        eqf = jnp.min(eqv.astype(jnp.float32))
        sums = tuple(jnp.where(eqv, sums[j] + vrow[j], vrow[j])
                     for j in range(8))
        maxs = tuple(jnp.where(eqv, jnp.maximum(maxs[j], vrow[j]), vrow[j])
                     for j in range(8))
        return sums, maxs, segr, cnt * eqf + 1.0

    def block_body(b, acc):
        want = r0 + b * BLOCKR
        base = jnp.minimum(want, N - BLOCKR)
        off = want - base
        pltpu.sync_copy(x_hbm.at[pl.ds(base, BLOCKR), :], xbuf)
        nch_b = jnp.minimum(CH, nch - b * CH)

        def chunk_body(c, acc):
            brow = off + c * 16
            idrow = idoff + (b * CH + c) * 16
            ids_vec = idbuf[pl.ds(idrow, 16)]
            _, _, runseg, _ = acc
            allsame = jnp.min((ids_vec == runseg).astype(jnp.int32)) == 1

            def fast(acc):
                for r in range(16):
                    vrow = tuple(xbuf[brow + r, pl.ds(j * 16, 16)]
                                 for j in range(8))
                    acc = row_fast(vrow, acc)
                return acc

            def slow(acc):
                for r in range(16):
                    segr = plsc.load_gather(
                        idbuf, [jnp.full((16,), idrow + r, jnp.int32)])
                    vrow = tuple(xbuf[brow + r, pl.ds(j * 16, 16)]
                                 for j in range(8))
                    acc = row_slow(segr, vrow, acc)
                return acc

            return lax.cond(allsame, fast, slow, acc)

        return lax.fori_loop(0, nch_b, chunk_body, acc)

    acc0 = (tuple(zero16 for _ in range(8)),
            tuple(jnp.full((16,), _NEG_INF, jnp.float32) for _ in range(8)),
            jnp.full((16,), -1, jnp.int32),
            jnp.float32(0.0))
    sums, maxs, runseg, cnt = lax.fori_loop(0, nbt, block_body, acc0)

    flush = runseg >= 0
    for j in range(8):
        cidx = col + (j * 16)
        plsc.store_scatter(sumtab, [runseg, cidx], sums[j], mask=flush)
        plsc.store_scatter(maxtab, [runseg, cidx], maxs[j], mask=flush)
    plsc.store_scatter(cnttab, [runseg, col], jnp.full((16,), cnt),
                       mask=flush)

    pltpu.sync_copy(sumtab, psum_hbm.at[wid])
    pltpu.sync_copy(maxtab, pmax_hbm.at[wid])
    pltpu.sync_copy(cnttab, pcnt_hbm.at[wid])


def _dot_t(a, b):
    return jax.lax.dot_general(a, b, (((1,), (1,)), ((), ())),
                               preferred_element_type=jnp.float32)


def _tc_tail(psum, pmax, pcnt, Wm, bm, Wx, bx, Ws, bs, Wgm, bgm, Wgx, bgx,
             Wgs, bgs, Wo, bo, gamma, beta, out_ref):
    ones16 = jnp.ones((D, 16), dtype=jnp.float32)
    sum_pool = jnp.zeros((S, D), jnp.float32)
    max_pool = jnp.full((S, D), _NEG_INF, jnp.float32)
    cnt2 = jnp.zeros((S, 16), jnp.float32)
    for i in range(NW):
        ci = pcnt[i]
        m = lax.dot_general(ci, ones16, (((1,), (1,)), ((), ())),
                            preferred_element_type=jnp.float32) > 0
        sum_pool = sum_pool + jnp.where(m, psum[i], 0.0)
        max_pool = jnp.where(m, jnp.maximum(max_pool, pmax[i]), max_pool)
        cnt2 = cnt2 + ci

    counts = lax.dot_general(cnt2, ones16, (((1,), (1,)), ((), ())),
                             preferred_element_type=jnp.float32) * (1.0 / 16.0)
    counts = jnp.maximum(counts, 1.0)
    mean_pool = sum_pool / counts

    mean_repr = _dot_t(mean_pool, Wm[...]) + bm[...]
    max_repr = _dot_t(max_pool, Wx[...]) + bx[...]
    sum_repr = _dot_t(sum_pool, Ws[...]) + bs[...]

    gm = _dot_t(mean_repr, Wgm[...]) + bgm[...]
    gx = _dot_t(max_repr, Wgx[...]) + bgx[...]
    gs = _dot_t(sum_repr, Wgs[...]) + bgs[...]
    gm = 1.0 / (1.0 + jnp.exp(-gm))
    gx = 1.0 / (1.0 + jnp.exp(-gx))
    gs = 1.0 / (1.0 + jnp.exp(-gs))

    mx = jnp.maximum(jnp.maximum(gm, gx), gs)
    em = jnp.exp(gm - mx)
    ex = jnp.exp(gx - mx)
    es = jnp.exp(gs - mx)
    denom = em + ex + es
    pooled = (em * mean_repr + ex * max_repr + es * sum_repr) / denom

    ge = _dot_t(pooled, Wo[...]) + bo[...]
    ones = jnp.ones((D, D), dtype=jnp.float32)
    mu = _dot_t(ge, ones) * (1.0 / D)
    dev = ge - mu
    var = _dot_t(dev * dev, ones) * (1.0 / D)
    out_ref[...] = dev / jnp.sqrt(var + EPS) * gamma[...] + beta[...]


def kernel(x, batch, Wm, bm, Wx, bx, Ws, bs, Wgm, bgm, Wgx, bgx,
           Wgs, bgs, Wo, bo, gamma, beta):
    ids = batch.astype(jnp.int32)

    mesh = plsc.VectorSubcoreMesh(core_axis_name="c", subcore_axis_name="s")
    pool = functools.partial(
        pl.kernel,
        mesh=mesh,
        out_type=[
            jax.ShapeDtypeStruct((NW, S, D), jnp.float32),
            jax.ShapeDtypeStruct((NW, S, D), jnp.float32),
            jax.ShapeDtypeStruct((NW, S, 16), jnp.float32),
        ],
        scratch_types=[
            pltpu.VMEM((BLOCKR, D), jnp.float32),
            pltpu.VMEM((ROWS_W,), jnp.int32),
            pltpu.VMEM((S, D), jnp.float32),
            pltpu.VMEM((S, D), jnp.float32),
            pltpu.VMEM((S, 16), jnp.float32),
        ],
        compiler_params=pltpu.CompilerParams(needs_layout_passes=False),
    )(_sc_pool)
    psum, pmax, pcnt = pool(x, ids)

    b2 = lambda b: b.reshape(1, -1)
    Wgm_r = jnp.broadcast_to(Wgm, (D, D))
    Wgx_r = jnp.broadcast_to(Wgx, (D, D))
    Wgs_r = jnp.broadcast_to(Wgs, (D, D))
    bgm_r = jnp.broadcast_to(bgm.reshape(1, 1), (1, D))
    bgx_r = jnp.broadcast_to(bgx.reshape(1, 1), (1, D))
    bgs_r = jnp.broadcast_to(bgs.reshape(1, 1), (1, D))

    args = (psum, pmax, pcnt, Wm, b2(bm), Wx, b2(bx), Ws, b2(bs),
            Wgm_r, bgm_r, Wgx_r, bgx_r, Wgs_r, bgs_r, Wo, b2(bo),
            b2(gamma), b2(beta))

    return pl.pallas_call(
        _tc_tail,
        out_shape=jax.ShapeDtypeStruct((S, D), jnp.float32),
    )(*args)


# speedup vs baseline: 4.3705x; 1.1690x over previous
import functools

import jax
import jax.numpy as jnp
from jax import lax
from jax.experimental import pallas as pl
from jax.experimental.pallas import tpu as pltpu
from jax.experimental.pallas import tpu_sc as plsc

N = 100000
D = 128
S = 256
EPS = 1e-5

NW = 32
ROWS_W = 3136
BLOCKR = 112
CH = BLOCKR // 16

_NEG_INF = float("-inf")


def _sc_pool(x_hbm, ids_hbm, psum_hbm, pmax_hbm, pcnt_hbm,
             xbuf, idbuf, sumtab, maxtab, cnttab, sem):
    wid = lax.axis_index("s") * 2 + lax.axis_index("c")
    col = lax.iota(jnp.int32, 16)
    zero16 = jnp.zeros((16,), jnp.float32)

    def initc(i, c):
        cnttab[i, :] = zero16
        return c

    lax.fori_loop(0, S, initc, 0)

    r0 = wid * ROWS_W
    r1 = jnp.minimum(r0 + ROWS_W, N)
    nch = (r1 - r0) // 16
    nbt = (nch + CH - 1) // CH

    idbase = jnp.minimum(r0, N - ROWS_W)
    idoff = r0 - idbase
    pltpu.sync_copy(ids_hbm.at[pl.ds(idbase, ROWS_W)], idbuf)

    def row_fast(vrow, acc):
        sums, maxs, runseg, cnt = acc
        sums = tuple(sums[j] + vrow[j] for j in range(8))
        maxs = tuple(jnp.maximum(maxs[j], vrow[j]) for j in range(8))
        return sums, maxs, runseg, cnt + 1.0

    def row_slow(segr, vrow, acc):
        sums, maxs, runseg, cnt = acc
        eqv = segr == runseg
        flush = jnp.logical_and(jnp.logical_not(eqv), runseg >= 0)
        for j in range(8):
            cidx = col + (j * 16)
            plsc.store_scatter(sumtab, [runseg, cidx], sums[j], mask=flush)
            plsc.store_scatter(maxtab, [runseg, cidx], maxs[j], mask=flush)
        plsc.store_scatter(cnttab, [runseg, col], jnp.full((16,), cnt),
                           mask=flush)
        eqf = jnp.min(eqv.astype(jnp.float32))
        sums = tuple(jnp.where(eqv, sums[j] + vrow[j], vrow[j])
                     for j in range(8))
        maxs = tuple(jnp.where(eqv, jnp.maximum(maxs[j], vrow[j]), vrow[j])
                     for j in range(8))
        return sums, maxs, segr, cnt * eqf + 1.0

    def xcopy(b, par):
        base = jnp.minimum(r0 + b * BLOCKR, N - BLOCKR)
        return pltpu.make_async_copy(
            x_hbm.at[pl.ds(base, BLOCKR), :], xbuf.at[par], sem.at[par])

    xcopy(0, 0).start()

    def block_body(b, acc):
        par = lax.rem(b, 2)
        want = r0 + b * BLOCKR
        base = jnp.minimum(want, N - BLOCKR)
        off = want - base
        xcopy(b, par).wait()

        @pl.when(b + 1 < nbt)
        def _prefetch():
            xcopy(b + 1, 1 - par).start()

        nch_b = jnp.minimum(CH, nch - b * CH)

        def chunk_body(c, acc):
            brow = off + c * 16
            idrow = idoff + (b * CH + c) * 16
            ids_vec = idbuf[pl.ds(idrow, 16)]
            _, _, runseg, _ = acc
            allsame = jnp.min((ids_vec == runseg).astype(jnp.int32)) == 1

            def fast(acc):
                for r in range(16):
                    vrow = tuple(xbuf[par, brow + r, pl.ds(j * 16, 16)]
                                 for j in range(8))
                    acc = row_fast(vrow, acc)
                return acc

            def slow(acc):
                for r in range(16):
                    segr = plsc.load_gather(
                        idbuf, [jnp.full((16,), idrow + r, jnp.int32)])
                    vrow = tuple(xbuf[par, brow + r, pl.ds(j * 16, 16)]
                                 for j in range(8))
                    acc = row_slow(segr, vrow, acc)
                return acc

            return lax.cond(allsame, fast, slow, acc)

        return lax.fori_loop(0, nch_b, chunk_body, acc)

    acc0 = (tuple(zero16 for _ in range(8)),
            tuple(jnp.full((16,), _NEG_INF, jnp.float32) for _ in range(8)),
            jnp.full((16,), -1, jnp.int32),
            jnp.float32(0.0))
    sums, maxs, runseg, cnt = lax.fori_loop(0, nbt, block_body, acc0)

    flush = runseg >= 0
    for j in range(8):
        cidx = col + (j * 16)
        plsc.store_scatter(sumtab, [runseg, cidx], sums[j], mask=flush)
        plsc.store_scatter(maxtab, [runseg, cidx], maxs[j], mask=flush)
    plsc.store_scatter(cnttab, [runseg, col], jnp.full((16,), cnt),
                       mask=flush)

    pltpu.sync_copy(sumtab, psum_hbm.at[wid])
    pltpu.sync_copy(maxtab, pmax_hbm.at[wid])
    pltpu.sync_copy(cnttab, pcnt_hbm.at[wid])


def _dot_t(a, b):
    return jax.lax.dot_general(a, b, (((1,), (1,)), ((), ())),
                               preferred_element_type=jnp.float32)


def _tc_tail(psum, pmax, pcnt, Wm, bm, Wx, bx, Ws, bs, Wgm, bgm, Wgx, bgx,
             Wgs, bgs, Wo, bo, gamma, beta, out_ref):
    ones16 = jnp.ones((D, 16), dtype=jnp.float32)
    sum_pool = jnp.zeros((S, D), jnp.float32)
    max_pool = jnp.full((S, D), _NEG_INF, jnp.float32)
    cnt2 = jnp.zeros((S, 16), jnp.float32)
    for i in range(NW):
        ci = pcnt[i]
        m = lax.dot_general(ci, ones16, (((1,), (1,)), ((), ())),
                            preferred_element_type=jnp.float32) > 0
        sum_pool = sum_pool + jnp.where(m, psum[i], 0.0)
        max_pool = jnp.where(m, jnp.maximum(max_pool, pmax[i]), max_pool)
        cnt2 = cnt2 + ci

    counts = lax.dot_general(cnt2, ones16, (((1,), (1,)), ((), ())),
                             preferred_element_type=jnp.float32) * (1.0 / 16.0)
    counts = jnp.maximum(counts, 1.0)
    mean_pool = sum_pool / counts

    mean_repr = _dot_t(mean_pool, Wm[...]) + bm[...]
    max_repr = _dot_t(max_pool, Wx[...]) + bx[...]
    sum_repr = _dot_t(sum_pool, Ws[...]) + bs[...]

    gm = _dot_t(mean_repr, Wgm[...]) + bgm[...]
    gx = _dot_t(max_repr, Wgx[...]) + bgx[...]
    gs = _dot_t(sum_repr, Wgs[...]) + bgs[...]
    gm = 1.0 / (1.0 + jnp.exp(-gm))
    gx = 1.0 / (1.0 + jnp.exp(-gx))
    gs = 1.0 / (1.0 + jnp.exp(-gs))

    mx = jnp.maximum(jnp.maximum(gm, gx), gs)
    em = jnp.exp(gm - mx)
    ex = jnp.exp(gx - mx)
    es = jnp.exp(gs - mx)
    denom = em + ex + es
    pooled = (em * mean_repr + ex * max_repr + es * sum_repr) / denom

    ge = _dot_t(pooled, Wo[...]) + bo[...]
    ones = jnp.ones((D, D), dtype=jnp.float32)
    mu = _dot_t(ge, ones) * (1.0 / D)
    dev = ge - mu
    var = _dot_t(dev * dev, ones) * (1.0 / D)
    out_ref[...] = dev / jnp.sqrt(var + EPS) * gamma[...] + beta[...]


def kernel(x, batch, Wm, bm, Wx, bx, Ws, bs, Wgm, bgm, Wgx, bgx,
           Wgs, bgs, Wo, bo, gamma, beta):
    ids = batch.astype(jnp.int32)

    mesh = plsc.VectorSubcoreMesh(core_axis_name="c", subcore_axis_name="s")
    pool = functools.partial(
        pl.kernel,
        mesh=mesh,
        out_type=[
            jax.ShapeDtypeStruct((NW, S, D), jnp.float32),
            jax.ShapeDtypeStruct((NW, S, D), jnp.float32),
            jax.ShapeDtypeStruct((NW, S, 16), jnp.float32),
        ],
        scratch_types=[
            pltpu.VMEM((2, BLOCKR, D), jnp.float32),
            pltpu.VMEM((ROWS_W,), jnp.int32),
            pltpu.VMEM((S, D), jnp.float32),
            pltpu.VMEM((S, D), jnp.float32),
            pltpu.VMEM((S, 16), jnp.float32),
            pltpu.SemaphoreType.DMA((2,)),
        ],
        compiler_params=pltpu.CompilerParams(needs_layout_passes=False),
    )(_sc_pool)
    psum, pmax, pcnt = pool(x, ids)

    b2 = lambda b: b.reshape(1, -1)
    Wgm_r = jnp.broadcast_to(Wgm, (D, D))
    Wgx_r = jnp.broadcast_to(Wgx, (D, D))
    Wgs_r = jnp.broadcast_to(Wgs, (D, D))
    bgm_r = jnp.broadcast_to(bgm.reshape(1, 1), (1, D))
    bgx_r = jnp.broadcast_to(bgx.reshape(1, 1), (1, D))
    bgs_r = jnp.broadcast_to(bgs.reshape(1, 1), (1, D))

    args = (psum, pmax, pcnt, Wm, b2(bm), Wx, b2(bx), Ws, b2(bs),
            Wgm_r, bgm_r, Wgx_r, bgx_r, Wgs_r, bgs_r, Wo, b2(bo),
            b2(gamma), b2(beta))

    return pl.pallas_call(
        _tc_tail,
        out_shape=jax.ShapeDtypeStruct((S, D), jnp.float32),
    )(*args)
